# Initial kernel scaffold; baseline (speedup 1.0000x reference)
import jax
import jax.numpy as jnp
from jax.experimental import pallas as pl


def kernel(x, edge_index, W1, b1, W2, b2):
    raise NotImplementedError("write your pallas kernel here")


# speedup vs baseline: 7.8904x; 7.8904x over previous
import functools

import jax
import jax.numpy as jnp
from jax import lax
from jax.experimental import pallas as pl
from jax.experimental.pallas import tpu as pltpu
from jax.experimental.pallas import tpu_sc as plsc

_N = 10000
_E = 160000
_IN = 256
_OUT = 256
_HID = 512

_NPAD = 10240
_EPAD = 163840
_NC = 2
_NS = 16
_F = 128
_CHUNK = 128


def _vmesh():
    return plsc.VectorSubcoreMesh(core_axis_name="c", subcore_axis_name="s")


def _sc_params():
    return pltpu.CompilerParams(needs_layout_passes=False)


def _deg_call(dst_pad):
    per_w = _EPAD // (_NC * _NS)
    n_chunks = per_w // _CHUNK
    wb = _NPAD // _NS

    @functools.partial(
        pl.kernel,
        out_type=jax.ShapeDtypeStruct((_NC * _NPAD, _F), jnp.float32),
        mesh=_vmesh(),
        scratch_types=[
            pltpu.VMEM((_CHUNK,), jnp.int32),
            pltpu.VMEM((_CHUNK, _F), jnp.float32),
            pltpu.VMEM((_CHUNK, _F), jnp.float32),
            pltpu.VMEM_SHARED((_NPAD, _F), jnp.float32),
        ],
        compiler_params=_sc_params(),
    )
    def k(dst_hbm, out_hbm, didx_v, ones_v, zbuf_v, hist_sh):
        c = lax.axis_index("c")
        s = lax.axis_index("s")

        @pl.loop(0, _CHUNK)
        def _(i):
            for g in range(_F // 16):
                ones_v[i, pl.ds(g * 16, 16)] = jnp.full((16,), 1.0,
                                                        jnp.float32)
                zbuf_v[i, pl.ds(g * 16, 16)] = jnp.zeros((16,), jnp.float32)

        for kk in range(wb // _CHUNK):
            pltpu.sync_copy(zbuf_v,
                            hist_sh.at[pl.ds(s * wb + kk * _CHUNK, _CHUNK)])
        plsc.subcore_barrier()

        base = (c * _NS + s) * per_w

        @pl.loop(0, n_chunks)
        def _(t):
            pltpu.sync_copy(dst_hbm.at[pl.ds(base + t * _CHUNK, _CHUNK)],
                            didx_v)
            pltpu.sync_copy(ones_v, hist_sh.at[didx_v], add=True)

        plsc.subcore_barrier()
        pltpu.sync_copy(hist_sh.at[pl.ds(s * wb, wb)],
                        out_hbm.at[pl.ds(c * _NPAD + s * wb, wb)])

    return k(dst_pad)


def _agg_call(xs_cat, src_pad, dst_pad):
    per_s = _EPAD // _NS
    n_chunks = per_s // _CHUNK
    wb = _NPAD // _NS

    @functools.partial(
        pl.kernel,
        out_type=jax.ShapeDtypeStruct((_NC * _NPAD, _F), jnp.float32),
        mesh=_vmesh(),
        scratch_types=[
            pltpu.VMEM((_CHUNK,), jnp.int32),
            pltpu.VMEM((_CHUNK,), jnp.int32),
            pltpu.VMEM((_CHUNK,), jnp.int32),
            pltpu.VMEM((_CHUNK, _F), jnp.float32),
            pltpu.VMEM((_CHUNK, _F), jnp.float32),
            pltpu.VMEM_SHARED((_NPAD, _F), jnp.float32),
            pltpu.SemaphoreType.DMA,
        ],
        compiler_params=_sc_params(),
    )
    def k(xs_hbm, src_hbm, dst_hbm, out_hbm,
          sidx_v, didx_v, gidx_v, rows_v, zbuf_v, acc_sh, sem):
        c = lax.axis_index("c")
        s = lax.axis_index("s")

        @pl.loop(0, _CHUNK)
        def _(i):
            for g in range(_F // 16):
                zbuf_v[i, pl.ds(g * 16, 16)] = jnp.zeros((16,), jnp.float32)

        for kk in range(wb // _CHUNK):
            pltpu.sync_copy(zbuf_v,
                            acc_sh.at[pl.ds(s * wb + kk * _CHUNK, _CHUNK)])
        plsc.subcore_barrier()

        base = s * per_s
        coff = c * _NPAD

        @pl.loop(0, n_chunks)
        def _(t):
            off = base + t * _CHUNK
            pltpu.sync_copy(src_hbm.at[pl.ds(off, _CHUNK)], sidx_v)
            pltpu.sync_copy(dst_hbm.at[pl.ds(off, _CHUNK)], didx_v)
            for g in range(_CHUNK // 16):
                gidx_v[pl.ds(g * 16, 16)] = sidx_v[pl.ds(g * 16, 16)] + coff
            pltpu.async_copy(xs_hbm.at[gidx_v], rows_v, sem).wait()
            pltpu.sync_copy(rows_v, acc_sh.at[didx_v], add=True)

        plsc.subcore_barrier()
        pltpu.sync_copy(acc_sh.at[pl.ds(s * wb, wb)],
                        out_hbm.at[pl.ds(coff + s * wb, wb)])

    return k(xs_cat, src_pad, dst_pad)


_R = 1024


def _dense_call(agg1, x_pad, dis_c, inv_c, W1, b1r, W2):
    def body(agg_ref, x_ref, dis_ref, inv_ref, w1_ref, b1_ref, w2_ref,
             xs2_ref, p_ref):
        agg = jnp.concatenate([agg_ref[0], agg_ref[1]], axis=1)
        dis = dis_ref[...]
        inv = inv_ref[...]
        z1 = dis * agg + inv * x_ref[...]
        h1 = jnp.maximum(
            jnp.dot(z1, w1_ref[...], preferred_element_type=jnp.float32)
            + b1_ref[...], 0.0)
        h2 = jnp.dot(h1, w2_ref[...], preferred_element_type=jnp.float32)
        xs2 = dis * h2
        xs2_ref[0] = xs2[:, :_F]
        xs2_ref[1] = xs2[:, _F:]
        p_ref[...] = inv * h2

    return pl.pallas_call(
        body,
        grid=(_NPAD // _R,),
        in_specs=[
            pl.BlockSpec((2, _R, _F), lambda i: (0, i, 0)),
            pl.BlockSpec((_R, _IN), lambda i: (i, 0)),
            pl.BlockSpec((_R, 1), lambda i: (i, 0)),
            pl.BlockSpec((_R, 1), lambda i: (i, 0)),
            pl.BlockSpec((_IN, _HID), lambda i: (0, 0)),
            pl.BlockSpec((1, _HID), lambda i: (0, 0)),
            pl.BlockSpec((_HID, _OUT), lambda i: (0, 0)),
        ],
        out_specs=[
            pl.BlockSpec((2, _R, _F), lambda i: (0, i, 0)),
            pl.BlockSpec((_R, _OUT), lambda i: (i, 0)),
        ],
        out_shape=[
            jax.ShapeDtypeStruct((2, _NPAD, _F), jnp.float32),
            jax.ShapeDtypeStruct((_NPAD, _OUT), jnp.float32),
        ],
    )(agg1, x_pad, dis_c, inv_c, W1, b1r, W2)


def _final_call(agg2, p, dis_c, b2r):
    def body(agg_ref, p_ref, dis_ref, b2_ref, o_ref):
        agg = jnp.concatenate([agg_ref[0], agg_ref[1]], axis=1)
        o_ref[...] = jnp.maximum(
            dis_ref[...] * agg + p_ref[...] + b2_ref[...], 0.0)

    return pl.pallas_call(
        body,
        grid=(_NPAD // _R,),
        in_specs=[
            pl.BlockSpec((2, _R, _F), lambda i: (0, i, 0)),
            pl.BlockSpec((_R, _OUT), lambda i: (i, 0)),
            pl.BlockSpec((_R, 1), lambda i: (i, 0)),
            pl.BlockSpec((1, _OUT), lambda i: (0, 0)),
        ],
        out_specs=pl.BlockSpec((_R, _OUT), lambda i: (i, 0)),
        out_shape=jax.ShapeDtypeStruct((_NPAD, _OUT), jnp.float32),
    )(agg2, p, dis_c, b2r)


def kernel(x, edge_index, W1, b1, W2, b2):
    src = edge_index[0]
    dst = edge_index[1]
    npad_e = _EPAD - _E
    src_pad = jnp.concatenate(
        [src, jnp.full((npad_e,), _N, jnp.int32)])
    dst_pad = jnp.concatenate(
        [dst, jnp.full((npad_e,), _NPAD - 1, jnp.int32)])
    x_pad = jnp.concatenate(
        [x, jnp.zeros((_NPAD - _N, _IN), x.dtype)], axis=0)

    deg_parts = _deg_call(dst_pad)
    deg = deg_parts[:_NPAD, 0] + deg_parts[_NPAD:, 0] + 1.0
    dis = lax.rsqrt(deg)
    inv = 1.0 / deg
    dis_c = dis[:, None]
    inv_c = inv[:, None]

    xs = dis_c * x_pad
    xs_cat = jnp.concatenate([xs[:, :_F], xs[:, _F:]], axis=0)

    agg1 = _agg_call(xs_cat, src_pad, dst_pad).reshape(2, _NPAD, _F)
    xs2_halves, p = _dense_call(agg1, x_pad, dis_c, inv_c,
                                W1, b1.reshape(1, -1), W2)
    agg2 = _agg_call(xs2_halves.reshape(2 * _NPAD, _F),
                     src_pad, dst_pad).reshape(2, _NPAD, _F)
    out = _final_call(agg2, p, dis_c, b2.reshape(1, -1))
    return out[:_N]


# speedup vs baseline: 10.3746x; 1.3148x over previous
import functools

import jax
import jax.numpy as jnp
from jax import lax
from jax.experimental import pallas as pl
from jax.experimental.pallas import tpu as pltpu
from jax.experimental.pallas import tpu_sc as plsc

_N = 10000
_E = 160000
_IN = 256
_OUT = 256
_HID = 512

_NPAD = 10240
_EPAD = 163840
_NC = 2
_NS = 16
_F = 128
_CHUNK = 128


def _vmesh():
    return plsc.VectorSubcoreMesh(core_axis_name="c", subcore_axis_name="s")


def _sc_params():
    return pltpu.CompilerParams(needs_layout_passes=False)


def _deg_call(dst2d):
    per_w = _EPAD // (_NC * _NS)
    n_chunks = per_w // _CHUNK
    wb = _NPAD // _NS

    @functools.partial(
        pl.kernel,
        out_type=jax.ShapeDtypeStruct((_NC * _NPAD, _F), jnp.float32),
        mesh=_vmesh(),
        scratch_types=[
            pltpu.VMEM((n_chunks, _CHUNK), jnp.int32),
            pltpu.VMEM((_CHUNK, _F), jnp.float32),
            pltpu.VMEM((_CHUNK, _F), jnp.float32),
            pltpu.VMEM_SHARED((_NPAD, _F), jnp.float32),
        ],
        compiler_params=_sc_params(),
    )
    def k(dst_hbm, out_hbm, didx_v, ones_v, zbuf_v, hist_sh):
        c = lax.axis_index("c")
        s = lax.axis_index("s")
        w = c * _NS + s

        pltpu.sync_copy(dst_hbm.at[pl.ds(w * n_chunks, n_chunks)], didx_v)

        @pl.loop(0, _CHUNK)
        def _(i):
            for g in range(_F // 16):
                ones_v[i, pl.ds(g * 16, 16)] = jnp.full((16,), 1.0,
                                                        jnp.float32)
                zbuf_v[i, pl.ds(g * 16, 16)] = jnp.zeros((16,), jnp.float32)

        for kk in range(wb // _CHUNK):
            pltpu.sync_copy(zbuf_v,
                            hist_sh.at[pl.ds(s * wb + kk * _CHUNK, _CHUNK)])
        plsc.subcore_barrier()

        @pl.loop(0, n_chunks)
        def _(t):
            pltpu.sync_copy(ones_v, hist_sh.at[didx_v.at[t]], add=True)

        plsc.subcore_barrier()
        pltpu.sync_copy(hist_sh.at[pl.ds(s * wb, wb)],
                        out_hbm.at[pl.ds(c * _NPAD + s * wb, wb)])

    return k(dst2d)


def _agg_call(xs_cat, src2d, dst2d):
    per_s = _EPAD // _NS
    n_chunks = per_s // _CHUNK
    nstage = n_chunks // 2
    wb = _NPAD // _NS

    @functools.partial(
        pl.kernel,
        out_type=jax.ShapeDtypeStruct((_NC * _NPAD, _F), jnp.float32),
        mesh=_vmesh(),
        scratch_types=[
            pltpu.VMEM((nstage, _CHUNK), jnp.int32),
            pltpu.VMEM((nstage, _CHUNK), jnp.int32),
            pltpu.VMEM((_CHUNK, _F), jnp.float32),
            pltpu.VMEM((_CHUNK, _F), jnp.float32),
            pltpu.VMEM_SHARED((_NPAD, _F), jnp.float32),
            pltpu.SemaphoreType.DMA,
            pltpu.SemaphoreType.DMA,
        ],
        compiler_params=_sc_params(),
    )
    def k(xs_hbm, src_hbm, dst_hbm, out_hbm,
          didx_v, gidx_v, rows0_v, rows1_v, acc_sh, sem0, sem1):
        c = lax.axis_index("c")
        s = lax.axis_index("s")
        coff = c * _NPAD

        @pl.loop(0, _CHUNK)
        def _(i):
            for g in range(_F // 16):
                rows0_v[i, pl.ds(g * 16, 16)] = jnp.zeros((16,), jnp.float32)

        for kk in range(wb // _CHUNK):
            pltpu.sync_copy(rows0_v,
                            acc_sh.at[pl.ds(s * wb + kk * _CHUNK, _CHUNK)])
        plsc.subcore_barrier()

        for phase in range(n_chunks // nstage):
            pbase = s * n_chunks + phase * nstage
            pltpu.sync_copy(src_hbm.at[pl.ds(pbase, nstage)], gidx_v)
            pltpu.sync_copy(dst_hbm.at[pl.ds(pbase, nstage)], didx_v)

            @pl.loop(0, nstage)
            def _(j):
                for g in range(_CHUNK // 16):
                    gidx_v[j, pl.ds(g * 16, 16)] = (
                        gidx_v[j, pl.ds(g * 16, 16)] + coff)

            pltpu.async_copy(xs_hbm.at[gidx_v.at[0]], rows0_v, sem0)

            @pl.loop(0, nstage // 2)
            def _(u):
                j0 = 2 * u
                j1 = 2 * u + 1
                pltpu.async_copy(xs_hbm.at[gidx_v.at[j1]], rows1_v, sem1)
                pltpu.make_async_copy(xs_hbm.at[gidx_v.at[j0]],
                                      rows0_v, sem0).wait()
                pltpu.sync_copy(rows0_v, acc_sh.at[didx_v.at[j0]], add=True)

                @pl.when(u < nstage // 2 - 1)
                def _():
                    pltpu.async_copy(xs_hbm.at[gidx_v.at[j1 + 1]],
                                     rows0_v, sem0)

                pltpu.make_async_copy(xs_hbm.at[gidx_v.at[j1]],
                                      rows1_v, sem1).wait()
                pltpu.sync_copy(rows1_v, acc_sh.at[didx_v.at[j1]], add=True)

        plsc.subcore_barrier()
        pltpu.sync_copy(acc_sh.at[pl.ds(s * wb, wb)],
                        out_hbm.at[pl.ds(coff + s * wb, wb)])

    return k(xs_cat, src2d, dst2d)


_R = 1024


def _dense_call(agg1, x_pad, dis_c, inv_c, W1, b1r, W2):
    def body(agg_ref, x_ref, dis_ref, inv_ref, w1_ref, b1_ref, w2_ref,
             xs2_ref, p_ref):
        agg = jnp.concatenate([agg_ref[0], agg_ref[1]], axis=1)
        dis = dis_ref[...]
        inv = inv_ref[...]
        z1 = dis * agg + inv * x_ref[...]
        h1 = jnp.maximum(
            jnp.dot(z1, w1_ref[...], preferred_element_type=jnp.float32)
            + b1_ref[...], 0.0)
        h2 = jnp.dot(h1, w2_ref[...], preferred_element_type=jnp.float32)
        xs2 = dis * h2
        xs2_ref[0] = xs2[:, :_F]
        xs2_ref[1] = xs2[:, _F:]
        p_ref[...] = inv * h2

    return pl.pallas_call(
        body,
        grid=(_NPAD // _R,),
        in_specs=[
            pl.BlockSpec((2, _R, _F), lambda i: (0, i, 0)),
            pl.BlockSpec((_R, _IN), lambda i: (i, 0)),
            pl.BlockSpec((_R, 1), lambda i: (i, 0)),
            pl.BlockSpec((_R, 1), lambda i: (i, 0)),
            pl.BlockSpec((_IN, _HID), lambda i: (0, 0)),
            pl.BlockSpec((1, _HID), lambda i: (0, 0)),
            pl.BlockSpec((_HID, _OUT), lambda i: (0, 0)),
        ],
        out_specs=[
            pl.BlockSpec((2, _R, _F), lambda i: (0, i, 0)),
            pl.BlockSpec((_R, _OUT), lambda i: (i, 0)),
        ],
        out_shape=[
            jax.ShapeDtypeStruct((2, _NPAD, _F), jnp.float32),
            jax.ShapeDtypeStruct((_NPAD, _OUT), jnp.float32),
        ],
    )(agg1, x_pad, dis_c, inv_c, W1, b1r, W2)


def _final_call(agg2, p, dis_c, b2r):
    def body(agg_ref, p_ref, dis_ref, b2_ref, o_ref):
        agg = jnp.concatenate([agg_ref[0], agg_ref[1]], axis=1)
        o_ref[...] = jnp.maximum(
            dis_ref[...] * agg + p_ref[...] + b2_ref[...], 0.0)

    return pl.pallas_call(
        body,
        grid=(_NPAD // _R,),
        in_specs=[
            pl.BlockSpec((2, _R, _F), lambda i: (0, i, 0)),
            pl.BlockSpec((_R, _OUT), lambda i: (i, 0)),
            pl.BlockSpec((_R, 1), lambda i: (i, 0)),
            pl.BlockSpec((1, _OUT), lambda i: (0, 0)),
        ],
        out_specs=pl.BlockSpec((_R, _OUT), lambda i: (i, 0)),
        out_shape=jax.ShapeDtypeStruct((_NPAD, _OUT), jnp.float32),
    )(agg2, p, dis_c, b2r)


def kernel(x, edge_index, W1, b1, W2, b2):
    src = edge_index[0]
    dst = edge_index[1]
    npad_e = _EPAD - _E
    src_pad = jnp.concatenate(
        [src, jnp.full((npad_e,), _N, jnp.int32)])
    dst_pad = jnp.concatenate(
        [dst, jnp.full((npad_e,), _NPAD - 1, jnp.int32)])
    x_pad = jnp.concatenate(
        [x, jnp.zeros((_NPAD - _N, _IN), x.dtype)], axis=0)
    src2d = src_pad.reshape(_EPAD // _CHUNK, _CHUNK)
    dst2d = dst_pad.reshape(_EPAD // _CHUNK, _CHUNK)

    deg_parts = _deg_call(dst2d)
    deg = deg_parts[:_NPAD, 0] + deg_parts[_NPAD:, 0] + 1.0
    dis = lax.rsqrt(deg)
    inv = 1.0 / deg
    dis_c = dis[:, None]
    inv_c = inv[:, None]

    xs = dis_c * x_pad
    xs_cat = jnp.concatenate([xs[:, :_F], xs[:, _F:]], axis=0)

    agg1 = _agg_call(xs_cat, src2d, dst2d).reshape(2, _NPAD, _F)
    xs2_halves, p = _dense_call(agg1, x_pad, dis_c, inv_c,
                                W1, b1.reshape(1, -1), W2)
    agg2 = _agg_call(xs2_halves.reshape(2 * _NPAD, _F),
                     src2d, dst2d).reshape(2, _NPAD, _F)
    out = _final_call(agg2, p, dis_c, b2.reshape(1, -1))
    return out[:_N]
